# baseline (speedup 1.0000x reference)
import functools

import jax
import jax.numpy as jnp
from jax import lax
from jax.experimental import pallas as pl
from jax.experimental.pallas import tpu as pltpu
from jax.experimental.pallas import tpu_sc as plsc

N = 100000
E = 1600000
H = 64
EMB = 128
G = 512

NC = 2
NS = 16
NW = NC * NS
EW = E // NW
SCE = 2000
NSC = EW // SCE
NGRP = SCE // 16

R = 7
RS = 16384
RSHIFT = 14
RMASK = RS - 1
NPAD = R * RS

STG = 1408
CAP = 51200

ROW_BLK = 5000

_mesh = plsc.VectorSubcoreMesh(
    core_axis_name="c", subcore_axis_name="s", num_cores=NC, num_subcores=NS
)


def _atom_body(x_ref, w_ref, b_ref, o_ref):
    o_ref[...] = jax.nn.relu(
        jnp.dot(x_ref[...], w_ref[...], preferred_element_type=jnp.float32)
        + b_ref[...]
    )


def _atom_mlp(x, w, b):
    n, f_in = x.shape
    h = w.shape[1]
    return pl.pallas_call(
        _atom_body,
        grid=(n // ROW_BLK,),
        in_specs=[
            pl.BlockSpec((ROW_BLK, f_in), lambda i: (i, 0)),
            pl.BlockSpec((f_in, h), lambda i: (0, 0)),
            pl.BlockSpec((1, h), lambda i: (0, 0)),
        ],
        out_specs=pl.BlockSpec((ROW_BLK, h), lambda i: (i, 0)),
        out_shape=jax.ShapeDtypeStruct((n, h), jnp.float32),
    )(x, w, b.reshape(1, h))


def _gin_body(h_ref, p_ref, w1_ref, b1_ref, w2_ref, b2_ref, o_ref):
    m = h_ref[...] + p_ref[0] + p_ref[1]
    t = jax.nn.relu(
        jnp.dot(m, w1_ref[...], preferred_element_type=jnp.float32) + b1_ref[...]
    )
    o_ref[...] = jax.nn.relu(
        jnp.dot(t, w2_ref[...], preferred_element_type=jnp.float32) + b2_ref[...]
    )


def _gin_mlp(h, partial, w1, b1, w2, b2):
    n = h.shape[0]
    return pl.pallas_call(
        _gin_body,
        grid=(n // ROW_BLK,),
        in_specs=[
            pl.BlockSpec((ROW_BLK, H), lambda i: (i, 0)),
            pl.BlockSpec((2, ROW_BLK, H), lambda i: (0, i, 0)),
            pl.BlockSpec((H, 2 * H), lambda i: (0, 0)),
            pl.BlockSpec((1, 2 * H), lambda i: (0, 0)),
            pl.BlockSpec((2 * H, H), lambda i: (0, 0)),
            pl.BlockSpec((1, H), lambda i: (0, 0)),
        ],
        out_specs=pl.BlockSpec((ROW_BLK, H), lambda i: (i, 0)),
        out_shape=jax.ShapeDtypeStruct((n, H), jnp.float32),
    )(h, partial, w1, b1.reshape(1, 2 * H), w2, b2.reshape(1, H))


def _final_body(s_ref, c_ref, wf1_ref, bf1_ref, wf2_ref, bf2_ref, o_ref):
    sums = s_ref[0] + s_ref[1]
    cnt = jnp.maximum(c_ref[0] + c_ref[1], 1.0)
    hg = sums / cnt[:, :1]
    t = jax.nn.relu(
        jnp.dot(hg, wf1_ref[...], preferred_element_type=jnp.float32)
        + bf1_ref[...]
    )
    o_ref[...] = (
        jnp.dot(t, wf2_ref[...], preferred_element_type=jnp.float32) + bf2_ref[...]
    )


def _final_mlp(sums_p, cnts_p, wf1, bf1, wf2, bf2):
    return pl.pallas_call(
        _final_body,
        out_shape=jax.ShapeDtypeStruct((G, EMB), jnp.float32),
    )(sums_p, cnts_p, wf1, bf1.reshape(1, H), wf2, bf2.reshape(1, EMB))


def _partition_body(src_hbm, dst_hbm, pslab, cnts, sbuf, dbuf, pstage, cntv):
    c = lax.axis_index("c")
    s = lax.axis_index("s")
    w = c * NS + s
    base = w * EW
    iota16 = lax.iota(jnp.int32, 16)
    dummy16 = jnp.full((16,), lax.shift_left(jnp.int32(RS), 17), jnp.int32)

    def superchunk(scx, carry):
        pltpu.sync_copy(src_hbm.at[pl.ds(base + scx * SCE, SCE)], sbuf)
        pltpu.sync_copy(dst_hbm.at[pl.ds(base + scx * SCE, SCE)], dbuf)

        def group(g, carry):
            bcnt, scnt = carry
            sv = sbuf[pl.ds(g * 16, 16)]
            dv = dbuf[pl.ds(g * 16, 16)]
            rv = lax.shift_right_logical(dv, RSHIFT)
            loc = jnp.bitwise_and(dv, RMASK)
            packed = jnp.bitwise_or(lax.shift_left(loc, 17), sv)
            _, psort = plsc.sort_key_val(rv, packed)
            start = jnp.int32(0)
            new_scnt = []
            for r in range(R):
                m = rv == r
                cr = plsc.all_reduce_population_count(m)[0]
                idx = jnp.bitwise_and(iota16 + start, 15)
                rot = lax.gather(
                    psort, idx[:, None],
                    lax.GatherDimensionNumbers(
                        offset_dims=(), collapsed_slice_dims=(0,),
                        start_index_map=(0,)),
                    (1,), mode=lax.GatherScatterMode.PROMISE_IN_BOUNDS)
                pstage[pl.ds(r * STG + scnt[r], 16)] = rot
                new_scnt.append(scnt[r] + cr)
                start = start + cr
            scnt = tuple(new_scnt)
            check = (g & 15) == 15
            new_b = []
            new_s = []
            for r in range(R):
                full = jnp.logical_and(check, scnt[r] >= 1024)

                @pl.when(full)
                def _flush(r=r, bc=bcnt[r]):
                    pltpu.sync_copy(
                        pstage.at[pl.ds(r * STG, 1024)],
                        pslab.at[pl.ds((w * R + r) * CAP + bc * 128, 1024)],
                    )
                    for k in range(16):
                        pstage[pl.ds(r * STG + k * 16, 16)] = pstage[pl.ds(r * STG + 1024 + k * 16, 16)]

                fi = full.astype(jnp.int32)
                new_b.append(bcnt[r] + 8 * fi)
                new_s.append(scnt[r] - 1024 * fi)
            return (tuple(new_b), tuple(new_s))

        return lax.fori_loop(0, NGRP, group, carry)

    zeros7 = tuple(jnp.int32(0) for _ in range(R))
    bcnt, scnt = lax.fori_loop(0, NSC, superchunk, (zeros7, zeros7))

    total = jnp.zeros((16,), jnp.int32)
    iota16b = lax.iota(jnp.int32, 16)
    for r in range(R):
        for k in range(8):
            pstage[pl.ds(r * STG + scnt[r] + k * 16, 16)] = dummy16
        pltpu.sync_copy(
            pstage.at[pl.ds(r * STG, STG)],
            pslab.at[pl.ds((w * R + r) * CAP + bcnt[r] * 128, STG)],
        )
        nblocks = bcnt[r] + (scnt[r] + 127) // 128
        total = jnp.where(iota16b == r, nblocks, total)
    cntv[...] = total
    pltpu.sync_copy(cntv, cnts.at[pl.ds(w * 16, 16)])


_partition = functools.partial(
    pl.kernel,
    out_type=(
        jax.ShapeDtypeStruct((NW * R * CAP,), jnp.int32),
        jax.ShapeDtypeStruct((NW * 16,), jnp.int32),
    ),
    mesh=_mesh,
    compiler_params=pltpu.CompilerParams(
        use_tc_tiling_on_sc=False, needs_layout_passes=False),
    scratch_types=[
        pltpu.VMEM((SCE,), jnp.int32),
        pltpu.VMEM((SCE,), jnp.int32),
        pltpu.VMEM((R * STG,), jnp.int32),
        pltpu.VMEM((16,), jnp.int32),
    ],
)(_partition_body)


DEPTH = 6


def _scatter_body(h_hbm, pslab, cnts, zeros_hbm, out_hbm, agg, pbuf, sidx, *rest):
    drefs = list(rest[:DEPTH])
    rows, zv, cntv = rest[DEPTH:DEPTH + 3]
    gsems = list(rest[DEPTH + 3:2 * DEPTH + 3])
    ssems = list(rest[2 * DEPTH + 3:3 * DEPTH + 3])
    c = lax.axis_index("c")
    s = lax.axis_index("s")
    w = c * NS + s
    pltpu.sync_copy(cnts.at[pl.ds(w * 16, 16)], cntv)
    pltpu.sync_copy(zeros_hbm, zv)
    cv = cntv[...]
    smask = jnp.int32((1 << 17) - 1)

    for r in range(R):
        for k in range(8):
            pltpu.async_copy(zv, agg.at[pl.ds(s * 1024 + k * 128, 128)], gsems[0])
        for k in range(8):
            pltpu.make_async_copy(zv, agg.at[pl.ds(s * 1024 + k * 128, 128)],
                                  gsems[0]).wait()
        plsc.subcore_barrier()

        nb = cv[r]
        njj = (nb + DEPTH - 1) // DEPTH
        sbase = (w * R + r) * CAP

        def jjb(jj, carry, sbase=sbase, nb=nb):
            pltpu.sync_copy(pslab.at[pl.ds(sbase + jj * (128 * DEPTH), 128 * DEPTH)], pbuf)
            for t in range(DEPTH):
                j = DEPTH * jj + t

                @pl.when(jnp.logical_and(jj > 0, j - DEPTH < nb))
                def _(t=t):
                    pltpu.make_async_copy(h_hbm.at[pl.ds(0, 128)],
                                          rows.at[t], ssems[t]).wait()

                @pl.when(j < nb)
                def _(t=t, j=j):
                    for kk in range(8):
                        v = pbuf[pl.ds(t * 128 + kk * 16, 16)]
                        sidx[pl.ds(t * 128 + kk * 16, 16)] = jnp.bitwise_and(v, smask)
                        drefs[t][pl.ds(kk * 16, 16)] = lax.shift_right_logical(v, 17)
                    pltpu.async_copy(h_hbm.at[sidx.at[pl.ds(t * 128, 128)]],
                                     rows.at[t], gsems[t])

            for t in range(DEPTH):
                j = DEPTH * jj + t

                @pl.when(j < nb)
                def _(t=t):
                    pltpu.make_async_copy(h_hbm.at[pl.ds(0, 128)],
                                          rows.at[t], gsems[t]).wait()
                    pltpu.async_copy(rows.at[t], agg.at[drefs[t]],
                                     ssems[t], add=True)

            return carry

        lax.fori_loop(0, njj, jjb, jnp.int32(0))
        nlast = njj - 1
        for t in range(DEPTH):
            @pl.when(jnp.logical_and(nlast >= 0, nlast * DEPTH + t < nb))
            def _(t=t):
                pltpu.make_async_copy(h_hbm.at[pl.ds(0, 128)],
                                      rows.at[t], ssems[t]).wait()
        plsc.subcore_barrier()

        for k in range(8):
            off = s * 1024 + k * 128
            pltpu.async_copy(
                agg.at[pl.ds(off, 128)], out_hbm.at[c, pl.ds(r * RS + off, 128)],
                gsems[1])
        for k in range(8):
            off = s * 1024 + k * 128
            pltpu.make_async_copy(
                agg.at[pl.ds(off, 128)], out_hbm.at[c, pl.ds(r * RS + off, 128)],
                gsems[1]).wait()


_scatter = functools.partial(
    pl.kernel,
    out_type=jax.ShapeDtypeStruct((NC, NPAD, H), jnp.float32),
    mesh=_mesh,
    compiler_params=pltpu.CompilerParams(use_tc_tiling_on_sc=False),
    scratch_types=[
        pltpu.VMEM_SHARED((RS + 16, H), jnp.float32),
        pltpu.VMEM((128 * DEPTH,), jnp.int32),
        pltpu.VMEM((128 * DEPTH,), jnp.int32),
    ] + [pltpu.VMEM((128,), jnp.int32) for _ in range(DEPTH)] + [
        pltpu.VMEM((DEPTH, 128, H), jnp.float32),
        pltpu.VMEM((128, H), jnp.float32),
        pltpu.VMEM((16,), jnp.int32),
    ] + [pltpu.SemaphoreType.DMA for _ in range(2 * DEPTH)],
)(_scatter_body)


NBLK = N // 128
NTAIL = N - NBLK * 128
NBLK_REM = NBLK - (NBLK // NW) * NW


def _pool_body(h_hbm, batch_hbm, ones_hbm, zeros_hbm, zeros16_hbm,
               sums_out, cnts_out,
               sums_sh, cnt_sh, rows, bidx, onesv, zv, z16v,
               rows32, bidx32, ones32v):
    c = lax.axis_index("c")
    s = lax.axis_index("s")
    w = c * NS + s

    pltpu.sync_copy(ones_hbm, onesv)
    pltpu.sync_copy(zeros_hbm, zv)
    pltpu.sync_copy(zeros16_hbm, z16v)

    @pl.when(s < 4)
    def _():
        pltpu.sync_copy(zv, sums_sh.at[pl.ds(s * 128, 128)])

    @pl.when(jnp.logical_and(s >= 4, s < 8))
    def _():
        pltpu.sync_copy(z16v, cnt_sh.at[pl.ds((s - 4) * 128, 128)])

    plsc.subcore_barrier()

    nj = jnp.where(w < NBLK_REM, NBLK // NW + 1, NBLK // NW)

    def blk(j, carry):
        b = w + j * NW
        pltpu.sync_copy(h_hbm.at[pl.ds(b * 128, 128)], rows)
        pltpu.sync_copy(batch_hbm.at[pl.ds(b * 128, 128)], bidx)
        pltpu.sync_copy(rows, sums_sh.at[bidx], add=True)
        pltpu.sync_copy(onesv, cnt_sh.at[bidx], add=True)
        return carry

    lax.fori_loop(0, nj, blk, jnp.int32(0))

    @pl.when(w == 0)
    def _():
        pltpu.sync_copy(h_hbm.at[pl.ds(NBLK * 128, NTAIL)], rows32)
        pltpu.sync_copy(batch_hbm.at[pl.ds(NBLK * 128, NTAIL)], bidx32)
        pltpu.sync_copy(ones_hbm.at[pl.ds(0, NTAIL)], ones32v)
        pltpu.sync_copy(rows32, sums_sh.at[bidx32], add=True)
        pltpu.sync_copy(ones32v, cnt_sh.at[bidx32], add=True)

    plsc.subcore_barrier()

    pltpu.sync_copy(sums_sh.at[pl.ds(s * 32, 32)], sums_out.at[c, pl.ds(s * 32, 32)])
    pltpu.sync_copy(cnt_sh.at[pl.ds(s * 32, 32)], cnts_out.at[c, pl.ds(s * 32, 32)])


_pool = functools.partial(
    pl.kernel,
    out_type=(
        jax.ShapeDtypeStruct((NC, G, H), jnp.float32),
        jax.ShapeDtypeStruct((NC, G, 16), jnp.float32),
    ),
    mesh=_mesh,
    compiler_params=pltpu.CompilerParams(use_tc_tiling_on_sc=False),
    scratch_types=[
        pltpu.VMEM_SHARED((G, H), jnp.float32),
        pltpu.VMEM_SHARED((G, 16), jnp.float32),
        pltpu.VMEM((128, H), jnp.float32),
        pltpu.VMEM((128,), jnp.int32),
        pltpu.VMEM((128, 16), jnp.float32),
        pltpu.VMEM((128, H), jnp.float32),
        pltpu.VMEM((128, 16), jnp.float32),
        pltpu.VMEM((NTAIL, H), jnp.float32),
        pltpu.VMEM((NTAIL,), jnp.int32),
        pltpu.VMEM((NTAIL, 16), jnp.float32),
    ],
)(_pool_body)


def kernel(x, edge_index, batch, W_atom, b_atom, W1, b1, W2, b2, Wf1, bf1, Wf2, bf2):
    src = edge_index[0].astype(jnp.int32)
    dst = edge_index[1].astype(jnp.int32)
    batch = batch.astype(jnp.int32)

    zeros64 = jnp.zeros((128, H), jnp.float32)
    zeros16 = jnp.zeros((128, 16), jnp.float32)
    ones16 = jnp.ones((128, 16), jnp.float32)

    h = _atom_mlp(x, W_atom, b_atom)
    pslab, cnts = _partition(src, dst)
    for i in range(4):
        partial = _scatter(h, pslab, cnts, zeros64)
        h = _gin_mlp(h, partial, W1[i], b1[i], W2[i], b2[i])
    sums_p, cnts_p = _pool(h, batch, ones16, zeros64, zeros16)
    return _final_mlp(sums_p, cnts_p, Wf1, bf1, Wf2, bf2)


# speedup vs baseline: 1.0266x; 1.0266x over previous
import functools

import jax
import jax.numpy as jnp
from jax import lax
from jax.experimental import pallas as pl
from jax.experimental.pallas import tpu as pltpu
from jax.experimental.pallas import tpu_sc as plsc

N = 100000
E = 1600000
H = 64
EMB = 128
G = 512

NC = 2
NS = 16
NW = NC * NS
EW = E // NW
SCE = 2000
NSC = EW // SCE
NGRP = SCE // 16

R = 7
RS = 16384
RSHIFT = 14
RMASK = RS - 1
NPAD = R * RS

STG = 1280
CAP = 51200

ROW_BLK = 5000

_mesh = plsc.VectorSubcoreMesh(
    core_axis_name="c", subcore_axis_name="s", num_cores=NC, num_subcores=NS
)


def _atom_body(x_ref, w_ref, b_ref, o_ref):
    o_ref[...] = jax.nn.relu(
        jnp.dot(x_ref[...], w_ref[...], preferred_element_type=jnp.float32)
        + b_ref[...]
    )


def _atom_mlp(x, w, b):
    n, f_in = x.shape
    h = w.shape[1]
    return pl.pallas_call(
        _atom_body,
        grid=(n // ROW_BLK,),
        in_specs=[
            pl.BlockSpec((ROW_BLK, f_in), lambda i: (i, 0)),
            pl.BlockSpec((f_in, h), lambda i: (0, 0)),
            pl.BlockSpec((1, h), lambda i: (0, 0)),
        ],
        out_specs=pl.BlockSpec((ROW_BLK, h), lambda i: (i, 0)),
        out_shape=jax.ShapeDtypeStruct((n, h), jnp.float32),
    )(x, w, b.reshape(1, h))


def _gin_body(h_ref, p_ref, w1_ref, b1_ref, w2_ref, b2_ref, o_ref):
    m = h_ref[...] + p_ref[0] + p_ref[1]
    t = jax.nn.relu(
        jnp.dot(m, w1_ref[...], preferred_element_type=jnp.float32) + b1_ref[...]
    )
    o_ref[...] = jax.nn.relu(
        jnp.dot(t, w2_ref[...], preferred_element_type=jnp.float32) + b2_ref[...]
    )


def _gin_mlp(h, partial, w1, b1, w2, b2):
    n = h.shape[0]
    return pl.pallas_call(
        _gin_body,
        grid=(n // ROW_BLK,),
        in_specs=[
            pl.BlockSpec((ROW_BLK, H), lambda i: (i, 0)),
            pl.BlockSpec((2, ROW_BLK, H), lambda i: (0, i, 0)),
            pl.BlockSpec((H, 2 * H), lambda i: (0, 0)),
            pl.BlockSpec((1, 2 * H), lambda i: (0, 0)),
            pl.BlockSpec((2 * H, H), lambda i: (0, 0)),
            pl.BlockSpec((1, H), lambda i: (0, 0)),
        ],
        out_specs=pl.BlockSpec((ROW_BLK, H), lambda i: (i, 0)),
        out_shape=jax.ShapeDtypeStruct((n, H), jnp.float32),
    )(h, partial, w1, b1.reshape(1, 2 * H), w2, b2.reshape(1, H))


def _final_body(s_ref, c_ref, wf1_ref, bf1_ref, wf2_ref, bf2_ref, o_ref):
    sums = s_ref[0] + s_ref[1]
    cnt = jnp.maximum(c_ref[0] + c_ref[1], 1.0)
    hg = sums / cnt[:, :1]
    t = jax.nn.relu(
        jnp.dot(hg, wf1_ref[...], preferred_element_type=jnp.float32)
        + bf1_ref[...]
    )
    o_ref[...] = (
        jnp.dot(t, wf2_ref[...], preferred_element_type=jnp.float32) + bf2_ref[...]
    )


def _final_mlp(sums_p, cnts_p, wf1, bf1, wf2, bf2):
    return pl.pallas_call(
        _final_body,
        out_shape=jax.ShapeDtypeStruct((G, EMB), jnp.float32),
    )(sums_p, cnts_p, wf1, bf1.reshape(1, H), wf2, bf2.reshape(1, EMB))


def _partition_body(src_hbm, dst_hbm, pslab, cnts, sbuf, dbuf, pstage, cntv):
    c = lax.axis_index("c")
    s = lax.axis_index("s")
    w = c * NS + s
    base = w * EW
    iota16 = lax.iota(jnp.int32, 16)
    dummy16 = jnp.full((16,), lax.shift_left(jnp.int32(RS), 17), jnp.int32)

    def superchunk(scx, carry):
        pltpu.sync_copy(src_hbm.at[pl.ds(base + scx * SCE, SCE)], sbuf)
        pltpu.sync_copy(dst_hbm.at[pl.ds(base + scx * SCE, SCE)], dbuf)

        def group(g, carry):
            bcnt, scnt = carry
            sv = sbuf[pl.ds(g * 16, 16)]
            dv = dbuf[pl.ds(g * 16, 16)]
            rv = lax.shift_right_logical(dv, RSHIFT)
            loc = jnp.bitwise_and(dv, RMASK)
            packed = jnp.bitwise_or(lax.shift_left(loc, 17), sv)
            _, psort = plsc.sort_key_val(rv, packed)
            start = jnp.int32(0)
            new_scnt = []
            for r in range(R):
                m = rv == r
                cr = plsc.all_reduce_population_count(m)[0]
                idx = jnp.bitwise_and(iota16 + start, 15)
                rot = lax.gather(
                    psort, idx[:, None],
                    lax.GatherDimensionNumbers(
                        offset_dims=(), collapsed_slice_dims=(0,),
                        start_index_map=(0,)),
                    (1,), mode=lax.GatherScatterMode.PROMISE_IN_BOUNDS)
                pstage[pl.ds(r * STG + scnt[r], 16)] = rot
                new_scnt.append(scnt[r] + cr)
                start = start + cr
            scnt = tuple(new_scnt)
            check = (g & 7) == 7
            new_b = []
            new_s = []
            for r in range(R):
                full = jnp.logical_and(check, scnt[r] >= 1024)

                @pl.when(full)
                def _flush(r=r, bc=bcnt[r]):
                    pltpu.sync_copy(
                        pstage.at[pl.ds(r * STG, 1024)],
                        pslab.at[pl.ds((w * R + r) * CAP + bc * 128, 1024)],
                    )
                    for k in range(8):
                        pstage[pl.ds(r * STG + k * 16, 16)] = pstage[pl.ds(r * STG + 1024 + k * 16, 16)]

                fi = full.astype(jnp.int32)
                new_b.append(bcnt[r] + 8 * fi)
                new_s.append(scnt[r] - 1024 * fi)
            return (tuple(new_b), tuple(new_s))

        return lax.fori_loop(0, NGRP, group, carry)

    zeros7 = tuple(jnp.int32(0) for _ in range(R))
    bcnt, scnt = lax.fori_loop(0, NSC, superchunk, (zeros7, zeros7))

    total = jnp.zeros((16,), jnp.int32)
    iota16b = lax.iota(jnp.int32, 16)
    for r in range(R):
        for k in range(8):
            pstage[pl.ds(r * STG + scnt[r] + k * 16, 16)] = dummy16
        pltpu.sync_copy(
            pstage.at[pl.ds(r * STG, STG)],
            pslab.at[pl.ds((w * R + r) * CAP + bcnt[r] * 128, STG)],
        )
        nblocks = bcnt[r] + (scnt[r] + 127) // 128
        total = jnp.where(iota16b == r, nblocks, total)
    cntv[...] = total
    pltpu.sync_copy(cntv, cnts.at[pl.ds(w * 16, 16)])


_partition = functools.partial(
    pl.kernel,
    out_type=(
        jax.ShapeDtypeStruct((NW * R * CAP,), jnp.int32),
        jax.ShapeDtypeStruct((NW * 16,), jnp.int32),
    ),
    mesh=_mesh,
    compiler_params=pltpu.CompilerParams(
        use_tc_tiling_on_sc=False, needs_layout_passes=False),
    scratch_types=[
        pltpu.VMEM((SCE,), jnp.int32),
        pltpu.VMEM((SCE,), jnp.int32),
        pltpu.VMEM((R * STG,), jnp.int32),
        pltpu.VMEM((16,), jnp.int32),
    ],
)(_partition_body)


DEPTH = 6


def _scatter_body(h_hbm, pslab, cnts, zeros_hbm, out_hbm, agg, pbuf, sidx, *rest):
    drefs = list(rest[:DEPTH])
    rows, zv, cntv = rest[DEPTH:DEPTH + 3]
    gsems = list(rest[DEPTH + 3:2 * DEPTH + 3])
    ssems = list(rest[2 * DEPTH + 3:3 * DEPTH + 3])
    c = lax.axis_index("c")
    s = lax.axis_index("s")
    w = c * NS + s
    pltpu.sync_copy(cnts.at[pl.ds(w * 16, 16)], cntv)
    pltpu.sync_copy(zeros_hbm, zv)
    cv = cntv[...]
    smask = jnp.int32((1 << 17) - 1)

    for r in range(R):
        for k in range(8):
            pltpu.async_copy(zv, agg.at[pl.ds(s * 1024 + k * 128, 128)], gsems[0])
        for k in range(8):
            pltpu.make_async_copy(zv, agg.at[pl.ds(s * 1024 + k * 128, 128)],
                                  gsems[0]).wait()
        plsc.subcore_barrier()

        nb = cv[r]
        njj = (nb + DEPTH - 1) // DEPTH
        sbase = (w * R + r) * CAP

        def jjb(jj, carry, sbase=sbase, nb=nb):
            pltpu.sync_copy(pslab.at[pl.ds(sbase + jj * (128 * DEPTH), 128 * DEPTH)], pbuf)
            for t in range(DEPTH):
                j = DEPTH * jj + t

                @pl.when(jnp.logical_and(jj > 0, j - DEPTH < nb))
                def _(t=t):
                    pltpu.make_async_copy(h_hbm.at[pl.ds(0, 128)],
                                          rows.at[t], ssems[t]).wait()

                @pl.when(j < nb)
                def _(t=t, j=j):
                    for kk in range(8):
                        v = pbuf[pl.ds(t * 128 + kk * 16, 16)]
                        sidx[pl.ds(t * 128 + kk * 16, 16)] = jnp.bitwise_and(v, smask)
                        drefs[t][pl.ds(kk * 16, 16)] = lax.shift_right_logical(v, 17)
                    pltpu.async_copy(h_hbm.at[sidx.at[pl.ds(t * 128, 128)]],
                                     rows.at[t], gsems[t])

            for t in range(DEPTH):
                j = DEPTH * jj + t

                @pl.when(j < nb)
                def _(t=t):
                    pltpu.make_async_copy(h_hbm.at[pl.ds(0, 128)],
                                          rows.at[t], gsems[t]).wait()
                    pltpu.async_copy(rows.at[t], agg.at[drefs[t]],
                                     ssems[t], add=True)

            return carry

        lax.fori_loop(0, njj, jjb, jnp.int32(0))
        nlast = njj - 1
        for t in range(DEPTH):
            @pl.when(jnp.logical_and(nlast >= 0, nlast * DEPTH + t < nb))
            def _(t=t):
                pltpu.make_async_copy(h_hbm.at[pl.ds(0, 128)],
                                      rows.at[t], ssems[t]).wait()
        plsc.subcore_barrier()

        for k in range(8):
            off = s * 1024 + k * 128
            pltpu.async_copy(
                agg.at[pl.ds(off, 128)], out_hbm.at[c, pl.ds(r * RS + off, 128)],
                gsems[1])
        for k in range(8):
            off = s * 1024 + k * 128
            pltpu.make_async_copy(
                agg.at[pl.ds(off, 128)], out_hbm.at[c, pl.ds(r * RS + off, 128)],
                gsems[1]).wait()


_scatter = functools.partial(
    pl.kernel,
    out_type=jax.ShapeDtypeStruct((NC, NPAD, H), jnp.float32),
    mesh=_mesh,
    compiler_params=pltpu.CompilerParams(use_tc_tiling_on_sc=False),
    scratch_types=[
        pltpu.VMEM_SHARED((RS + 16, H), jnp.float32),
        pltpu.VMEM((128 * DEPTH,), jnp.int32),
        pltpu.VMEM((128 * DEPTH,), jnp.int32),
    ] + [pltpu.VMEM((128,), jnp.int32) for _ in range(DEPTH)] + [
        pltpu.VMEM((DEPTH, 128, H), jnp.float32),
        pltpu.VMEM((128, H), jnp.float32),
        pltpu.VMEM((16,), jnp.int32),
    ] + [pltpu.SemaphoreType.DMA for _ in range(2 * DEPTH)],
)(_scatter_body)


NBLK = N // 128
NTAIL = N - NBLK * 128
NBLK_REM = NBLK - (NBLK // NW) * NW


def _pool_body(h_hbm, batch_hbm, ones_hbm, zeros_hbm, zeros16_hbm,
               sums_out, cnts_out,
               sums_sh, cnt_sh, rows, bidx, onesv, zv, z16v,
               rows32, bidx32, ones32v):
    c = lax.axis_index("c")
    s = lax.axis_index("s")
    w = c * NS + s

    pltpu.sync_copy(ones_hbm, onesv)
    pltpu.sync_copy(zeros_hbm, zv)
    pltpu.sync_copy(zeros16_hbm, z16v)

    @pl.when(s < 4)
    def _():
        pltpu.sync_copy(zv, sums_sh.at[pl.ds(s * 128, 128)])

    @pl.when(jnp.logical_and(s >= 4, s < 8))
    def _():
        pltpu.sync_copy(z16v, cnt_sh.at[pl.ds((s - 4) * 128, 128)])

    plsc.subcore_barrier()

    nj = jnp.where(w < NBLK_REM, NBLK // NW + 1, NBLK // NW)

    def blk(j, carry):
        b = w + j * NW
        pltpu.sync_copy(h_hbm.at[pl.ds(b * 128, 128)], rows)
        pltpu.sync_copy(batch_hbm.at[pl.ds(b * 128, 128)], bidx)
        pltpu.sync_copy(rows, sums_sh.at[bidx], add=True)
        pltpu.sync_copy(onesv, cnt_sh.at[bidx], add=True)
        return carry

    lax.fori_loop(0, nj, blk, jnp.int32(0))

    @pl.when(w == 0)
    def _():
        pltpu.sync_copy(h_hbm.at[pl.ds(NBLK * 128, NTAIL)], rows32)
        pltpu.sync_copy(batch_hbm.at[pl.ds(NBLK * 128, NTAIL)], bidx32)
        pltpu.sync_copy(ones_hbm.at[pl.ds(0, NTAIL)], ones32v)
        pltpu.sync_copy(rows32, sums_sh.at[bidx32], add=True)
        pltpu.sync_copy(ones32v, cnt_sh.at[bidx32], add=True)

    plsc.subcore_barrier()

    pltpu.sync_copy(sums_sh.at[pl.ds(s * 32, 32)], sums_out.at[c, pl.ds(s * 32, 32)])
    pltpu.sync_copy(cnt_sh.at[pl.ds(s * 32, 32)], cnts_out.at[c, pl.ds(s * 32, 32)])


_pool = functools.partial(
    pl.kernel,
    out_type=(
        jax.ShapeDtypeStruct((NC, G, H), jnp.float32),
        jax.ShapeDtypeStruct((NC, G, 16), jnp.float32),
    ),
    mesh=_mesh,
    compiler_params=pltpu.CompilerParams(use_tc_tiling_on_sc=False),
    scratch_types=[
        pltpu.VMEM_SHARED((G, H), jnp.float32),
        pltpu.VMEM_SHARED((G, 16), jnp.float32),
        pltpu.VMEM((128, H), jnp.float32),
        pltpu.VMEM((128,), jnp.int32),
        pltpu.VMEM((128, 16), jnp.float32),
        pltpu.VMEM((128, H), jnp.float32),
        pltpu.VMEM((128, 16), jnp.float32),
        pltpu.VMEM((NTAIL, H), jnp.float32),
        pltpu.VMEM((NTAIL,), jnp.int32),
        pltpu.VMEM((NTAIL, 16), jnp.float32),
    ],
)(_pool_body)


def kernel(x, edge_index, batch, W_atom, b_atom, W1, b1, W2, b2, Wf1, bf1, Wf2, bf2):
    src = edge_index[0].astype(jnp.int32)
    dst = edge_index[1].astype(jnp.int32)
    batch = batch.astype(jnp.int32)

    zeros64 = jnp.zeros((128, H), jnp.float32)
    zeros16 = jnp.zeros((128, 16), jnp.float32)
    ones16 = jnp.ones((128, 16), jnp.float32)

    h = _atom_mlp(x, W_atom, b_atom)
    pslab, cnts = _partition(src, dst)
    for i in range(4):
        partial = _scatter(h, pslab, cnts, zeros64)
        h = _gin_mlp(h, partial, W1[i], b1[i], W2[i], b2[i])
    sums_p, cnts_p = _pool(h, batch, ones16, zeros64, zeros16)
    return _final_mlp(sums_p, cnts_p, Wf1, bf1, Wf2, bf2)


# speedup vs baseline: 1.0424x; 1.0154x over previous
import functools

import jax
import jax.numpy as jnp
from jax import lax
from jax.experimental import pallas as pl
from jax.experimental.pallas import tpu as pltpu
from jax.experimental.pallas import tpu_sc as plsc

N = 100000
E = 1600000
H = 64
EMB = 128
G = 512

NC = 2
NS = 16
NW = NC * NS
EW = E // NW
SCE = 2000
NSC = EW // SCE
NGRP = SCE // 16

R = 7
RS = 16384
RSHIFT = 14
RMASK = RS - 1
NPAD = R * RS

STG = 1280
CAP = 51200

ROW_BLK = 5000

_mesh = plsc.VectorSubcoreMesh(
    core_axis_name="c", subcore_axis_name="s", num_cores=NC, num_subcores=NS
)


def _atom_body(x_ref, w_ref, b_ref, o_ref):
    o_ref[...] = jax.nn.relu(
        jnp.dot(x_ref[...], w_ref[...], preferred_element_type=jnp.float32)
        + b_ref[...]
    )


def _atom_mlp(x, w, b):
    n, f_in = x.shape
    h = w.shape[1]
    return pl.pallas_call(
        _atom_body,
        grid=(n // ROW_BLK,),
        in_specs=[
            pl.BlockSpec((ROW_BLK, f_in), lambda i: (i, 0)),
            pl.BlockSpec((f_in, h), lambda i: (0, 0)),
            pl.BlockSpec((1, h), lambda i: (0, 0)),
        ],
        out_specs=pl.BlockSpec((ROW_BLK, h), lambda i: (i, 0)),
        out_shape=jax.ShapeDtypeStruct((n, h), jnp.float32),
    )(x, w, b.reshape(1, h))


def _gin_body(h_ref, p_ref, w1_ref, b1_ref, w2_ref, b2_ref, o_ref):
    m = h_ref[...] + p_ref[0] + p_ref[1]
    t = jax.nn.relu(
        jnp.dot(m, w1_ref[...], preferred_element_type=jnp.float32) + b1_ref[...]
    )
    o_ref[...] = jax.nn.relu(
        jnp.dot(t, w2_ref[...], preferred_element_type=jnp.float32) + b2_ref[...]
    )


def _gin_mlp(h, partial, w1, b1, w2, b2):
    n = h.shape[0]
    return pl.pallas_call(
        _gin_body,
        grid=(n // ROW_BLK,),
        in_specs=[
            pl.BlockSpec((ROW_BLK, H), lambda i: (i, 0)),
            pl.BlockSpec((2, ROW_BLK, H), lambda i: (0, i, 0)),
            pl.BlockSpec((H, 2 * H), lambda i: (0, 0)),
            pl.BlockSpec((1, 2 * H), lambda i: (0, 0)),
            pl.BlockSpec((2 * H, H), lambda i: (0, 0)),
            pl.BlockSpec((1, H), lambda i: (0, 0)),
        ],
        out_specs=pl.BlockSpec((ROW_BLK, H), lambda i: (i, 0)),
        out_shape=jax.ShapeDtypeStruct((n, H), jnp.float32),
    )(h, partial, w1, b1.reshape(1, 2 * H), w2, b2.reshape(1, H))


def _final_body(s_ref, c_ref, wf1_ref, bf1_ref, wf2_ref, bf2_ref, o_ref):
    sums = s_ref[0] + s_ref[1]
    cnt = jnp.maximum(c_ref[0] + c_ref[1], 1.0)
    hg = sums / cnt[:, :1]
    t = jax.nn.relu(
        jnp.dot(hg, wf1_ref[...], preferred_element_type=jnp.float32)
        + bf1_ref[...]
    )
    o_ref[...] = (
        jnp.dot(t, wf2_ref[...], preferred_element_type=jnp.float32) + bf2_ref[...]
    )


def _final_mlp(sums_p, cnts_p, wf1, bf1, wf2, bf2):
    return pl.pallas_call(
        _final_body,
        out_shape=jax.ShapeDtypeStruct((G, EMB), jnp.float32),
    )(sums_p, cnts_p, wf1, bf1.reshape(1, H), wf2, bf2.reshape(1, EMB))


def _partition_body(src_hbm, dst_hbm, pslab, cnts, sbuf, dbuf, pstage, cntv):
    c = lax.axis_index("c")
    s = lax.axis_index("s")
    w = c * NS + s
    base = w * EW
    iota16 = lax.iota(jnp.int32, 16)
    dummy16 = jnp.full((16,), lax.shift_left(jnp.int32(RS), 17), jnp.int32)

    def superchunk(scx, carry):
        pltpu.sync_copy(src_hbm.at[pl.ds(base + scx * SCE, SCE)], sbuf)
        pltpu.sync_copy(dst_hbm.at[pl.ds(base + scx * SCE, SCE)], dbuf)

        def group(g, carry):
            bcnt, scnt = carry
            sv = sbuf[pl.ds(g * 16, 16)]
            dv = dbuf[pl.ds(g * 16, 16)]
            rv = lax.shift_right_logical(dv, RSHIFT)
            loc = jnp.bitwise_and(dv, RMASK)
            packed = jnp.bitwise_or(lax.shift_left(loc, 17), sv)
            _, psort = plsc.sort_key_val(rv, packed)
            start = jnp.int32(0)
            new_scnt = []
            for r in range(R):
                m = rv == r
                cr = plsc.all_reduce_population_count(m)[0]
                idx = jnp.bitwise_and(iota16 + start, 15)
                rot = lax.gather(
                    psort, idx[:, None],
                    lax.GatherDimensionNumbers(
                        offset_dims=(), collapsed_slice_dims=(0,),
                        start_index_map=(0,)),
                    (1,), mode=lax.GatherScatterMode.PROMISE_IN_BOUNDS)
                pstage[pl.ds(r * STG + scnt[r], 16)] = rot
                new_scnt.append(scnt[r] + cr)
                start = start + cr
            scnt = tuple(new_scnt)
            check = (g & 7) == 7
            new_b = []
            new_s = []
            for r in range(R):
                full = jnp.logical_and(check, scnt[r] >= 1024)

                @pl.when(full)
                def _flush(r=r, bc=bcnt[r]):
                    pltpu.sync_copy(
                        pstage.at[pl.ds(r * STG, 1024)],
                        pslab.at[pl.ds((w * R + r) * CAP + bc * 128, 1024)],
                    )
                    for k in range(8):
                        pstage[pl.ds(r * STG + k * 16, 16)] = pstage[pl.ds(r * STG + 1024 + k * 16, 16)]

                fi = full.astype(jnp.int32)
                new_b.append(bcnt[r] + 8 * fi)
                new_s.append(scnt[r] - 1024 * fi)
            return (tuple(new_b), tuple(new_s))

        return lax.fori_loop(0, NGRP, group, carry)

    zeros7 = tuple(jnp.int32(0) for _ in range(R))
    bcnt, scnt = lax.fori_loop(0, NSC, superchunk, (zeros7, zeros7))

    total = jnp.zeros((16,), jnp.int32)
    iota16b = lax.iota(jnp.int32, 16)
    for r in range(R):
        for k in range(8):
            pstage[pl.ds(r * STG + scnt[r] + k * 16, 16)] = dummy16
        pltpu.sync_copy(
            pstage.at[pl.ds(r * STG, STG)],
            pslab.at[pl.ds((w * R + r) * CAP + bcnt[r] * 128, STG)],
        )
        nblocks = bcnt[r] + (scnt[r] + 127) // 128
        total = jnp.where(iota16b == r, nblocks, total)
    cntv[...] = total
    pltpu.sync_copy(cntv, cnts.at[pl.ds(w * 16, 16)])


_partition = functools.partial(
    pl.kernel,
    out_type=(
        jax.ShapeDtypeStruct((NW * R * CAP,), jnp.int32),
        jax.ShapeDtypeStruct((NW * 16,), jnp.int32),
    ),
    mesh=_mesh,
    compiler_params=pltpu.CompilerParams(
        use_tc_tiling_on_sc=False, needs_layout_passes=False),
    scratch_types=[
        pltpu.VMEM((SCE,), jnp.int32),
        pltpu.VMEM((SCE,), jnp.int32),
        pltpu.VMEM((R * STG,), jnp.int32),
        pltpu.VMEM((16,), jnp.int32),
    ],
)(_partition_body)


DEPTH = 6


DEPTH = 6
GW = 128 * DEPTH


def _scatter_body(h_hbm, pslab, cnts, zeros_hbm, out_hbm, agg, pbufa, pbufb,
                  sidx, *rest):
    drefs = list(rest[:DEPTH])
    rows, zv, cntv = rest[DEPTH:DEPTH + 3]
    gsems = list(rest[DEPTH + 3:2 * DEPTH + 3])
    ssems = list(rest[2 * DEPTH + 3:3 * DEPTH + 3])
    psema, psemb = rest[3 * DEPTH + 3:3 * DEPTH + 5]
    c = lax.axis_index("c")
    s = lax.axis_index("s")
    w = c * NS + s
    pltpu.sync_copy(cnts.at[pl.ds(w * 16, 16)], cntv)
    pltpu.sync_copy(zeros_hbm, zv)
    cv = cntv[...]
    smask = jnp.int32((1 << 17) - 1)

    for r in range(R):
        for k in range(8):
            pltpu.async_copy(zv, agg.at[pl.ds(s * 1024 + k * 128, 128)], gsems[0])
        for k in range(8):
            pltpu.make_async_copy(zv, agg.at[pl.ds(s * 1024 + k * 128, 128)],
                                  gsems[0]).wait()
        plsc.subcore_barrier()

        nb = cv[r]
        njj = (nb + DEPTH - 1) // DEPTH
        sbase = (w * R + r) * CAP

        def process(g, pbuf, nb):
            for t in range(DEPTH):
                j = DEPTH * g + t

                @pl.when(jnp.logical_and(g > 0, j - DEPTH < nb))
                def _(t=t):
                    pltpu.make_async_copy(h_hbm.at[pl.ds(0, 128)],
                                          rows.at[t], ssems[t]).wait()

                @pl.when(j < nb)
                def _(t=t, j=j):
                    for kk in range(8):
                        v = pbuf[pl.ds(t * 128 + kk * 16, 16)]
                        sidx[pl.ds(t * 128 + kk * 16, 16)] = jnp.bitwise_and(v, smask)
                        drefs[t][pl.ds(kk * 16, 16)] = lax.shift_right_logical(v, 17)
                    pltpu.async_copy(h_hbm.at[sidx.at[pl.ds(t * 128, 128)]],
                                     rows.at[t], gsems[t])

            for t in range(DEPTH):
                j = DEPTH * g + t

                @pl.when(j < nb)
                def _(t=t):
                    pltpu.make_async_copy(h_hbm.at[pl.ds(0, 128)],
                                          rows.at[t], gsems[t]).wait()
                    pltpu.async_copy(rows.at[t], agg.at[drefs[t]],
                                     ssems[t], add=True)

        @pl.when(njj > 0)
        def _(sbase=sbase):
            pltpu.async_copy(pslab.at[pl.ds(sbase, GW)], pbufa, psema)

        def jjb(jjj, carry, sbase=sbase, nb=nb, njj=njj):
            ga = 2 * jjj
            gb = ga + 1

            @pl.when(gb < njj)
            def _():
                pltpu.async_copy(pslab.at[pl.ds(sbase + gb * GW, GW)], pbufb, psemb)

            pltpu.make_async_copy(pslab.at[pl.ds(sbase, GW)], pbufa, psema).wait()
            process(ga, pbufa, nb)

            @pl.when(gb + 1 < njj)
            def _():
                pltpu.async_copy(pslab.at[pl.ds(sbase + (gb + 1) * GW, GW)],
                                 pbufa, psema)

            @pl.when(gb < njj)
            def _():
                pltpu.make_async_copy(pslab.at[pl.ds(sbase, GW)], pbufb, psemb).wait()
                process(gb, pbufb, nb)

            return carry

        lax.fori_loop(0, (njj + 1) // 2, jjb, jnp.int32(0))
        nlast = njj - 1
        for t in range(DEPTH):
            @pl.when(jnp.logical_and(nlast >= 0, nlast * DEPTH + t < nb))
            def _(t=t):
                pltpu.make_async_copy(h_hbm.at[pl.ds(0, 128)],
                                      rows.at[t], ssems[t]).wait()
        plsc.subcore_barrier()

        for k in range(8):
            off = s * 1024 + k * 128
            pltpu.async_copy(
                agg.at[pl.ds(off, 128)], out_hbm.at[c, pl.ds(r * RS + off, 128)],
                gsems[1])
        for k in range(8):
            off = s * 1024 + k * 128
            pltpu.make_async_copy(
                agg.at[pl.ds(off, 128)], out_hbm.at[c, pl.ds(r * RS + off, 128)],
                gsems[1]).wait()


_scatter = functools.partial(
    pl.kernel,
    out_type=jax.ShapeDtypeStruct((NC, NPAD, H), jnp.float32),
    mesh=_mesh,
    compiler_params=pltpu.CompilerParams(use_tc_tiling_on_sc=False),
    scratch_types=[
        pltpu.VMEM_SHARED((RS + 16, H), jnp.float32),
        pltpu.VMEM((GW,), jnp.int32),
        pltpu.VMEM((GW,), jnp.int32),
        pltpu.VMEM((GW,), jnp.int32),
    ] + [pltpu.VMEM((128,), jnp.int32) for _ in range(DEPTH)] + [
        pltpu.VMEM((DEPTH, 128, H), jnp.float32),
        pltpu.VMEM((128, H), jnp.float32),
        pltpu.VMEM((16,), jnp.int32),
    ] + [pltpu.SemaphoreType.DMA for _ in range(2 * DEPTH + 2)],
)(_scatter_body)


NBLK = N // 128
NTAIL = N - NBLK * 128
NBLK_REM = NBLK - (NBLK // NW) * NW


def _pool_body(h_hbm, batch_hbm, ones_hbm, zeros_hbm, zeros16_hbm,
               sums_out, cnts_out,
               sums_sh, cnt_sh, rows, bidx, onesv, zv, z16v,
               rows32, bidx32, ones32v):
    c = lax.axis_index("c")
    s = lax.axis_index("s")
    w = c * NS + s

    pltpu.sync_copy(ones_hbm, onesv)
    pltpu.sync_copy(zeros_hbm, zv)
    pltpu.sync_copy(zeros16_hbm, z16v)

    @pl.when(s < 4)
    def _():
        pltpu.sync_copy(zv, sums_sh.at[pl.ds(s * 128, 128)])

    @pl.when(jnp.logical_and(s >= 4, s < 8))
    def _():
        pltpu.sync_copy(z16v, cnt_sh.at[pl.ds((s - 4) * 128, 128)])

    plsc.subcore_barrier()

    nj = jnp.where(w < NBLK_REM, NBLK // NW + 1, NBLK // NW)

    def blk(j, carry):
        b = w + j * NW
        pltpu.sync_copy(h_hbm.at[pl.ds(b * 128, 128)], rows)
        pltpu.sync_copy(batch_hbm.at[pl.ds(b * 128, 128)], bidx)
        pltpu.sync_copy(rows, sums_sh.at[bidx], add=True)
        pltpu.sync_copy(onesv, cnt_sh.at[bidx], add=True)
        return carry

    lax.fori_loop(0, nj, blk, jnp.int32(0))

    @pl.when(w == 0)
    def _():
        pltpu.sync_copy(h_hbm.at[pl.ds(NBLK * 128, NTAIL)], rows32)
        pltpu.sync_copy(batch_hbm.at[pl.ds(NBLK * 128, NTAIL)], bidx32)
        pltpu.sync_copy(ones_hbm.at[pl.ds(0, NTAIL)], ones32v)
        pltpu.sync_copy(rows32, sums_sh.at[bidx32], add=True)
        pltpu.sync_copy(ones32v, cnt_sh.at[bidx32], add=True)

    plsc.subcore_barrier()

    pltpu.sync_copy(sums_sh.at[pl.ds(s * 32, 32)], sums_out.at[c, pl.ds(s * 32, 32)])
    pltpu.sync_copy(cnt_sh.at[pl.ds(s * 32, 32)], cnts_out.at[c, pl.ds(s * 32, 32)])


_pool = functools.partial(
    pl.kernel,
    out_type=(
        jax.ShapeDtypeStruct((NC, G, H), jnp.float32),
        jax.ShapeDtypeStruct((NC, G, 16), jnp.float32),
    ),
    mesh=_mesh,
    compiler_params=pltpu.CompilerParams(use_tc_tiling_on_sc=False),
    scratch_types=[
        pltpu.VMEM_SHARED((G, H), jnp.float32),
        pltpu.VMEM_SHARED((G, 16), jnp.float32),
        pltpu.VMEM((128, H), jnp.float32),
        pltpu.VMEM((128,), jnp.int32),
        pltpu.VMEM((128, 16), jnp.float32),
        pltpu.VMEM((128, H), jnp.float32),
        pltpu.VMEM((128, 16), jnp.float32),
        pltpu.VMEM((NTAIL, H), jnp.float32),
        pltpu.VMEM((NTAIL,), jnp.int32),
        pltpu.VMEM((NTAIL, 16), jnp.float32),
    ],
)(_pool_body)


def kernel(x, edge_index, batch, W_atom, b_atom, W1, b1, W2, b2, Wf1, bf1, Wf2, bf2):
    src = edge_index[0].astype(jnp.int32)
    dst = edge_index[1].astype(jnp.int32)
    batch = batch.astype(jnp.int32)

    zeros64 = jnp.zeros((128, H), jnp.float32)
    zeros16 = jnp.zeros((128, 16), jnp.float32)
    ones16 = jnp.ones((128, 16), jnp.float32)

    h = _atom_mlp(x, W_atom, b_atom)
    pslab, cnts = _partition(src, dst)
    for i in range(4):
        partial = _scatter(h, pslab, cnts, zeros64)
        h = _gin_mlp(h, partial, W1[i], b1[i], W2[i], b2[i])
    sums_p, cnts_p = _pool(h, batch, ones16, zeros64, zeros16)
    return _final_mlp(sums_p, cnts_p, Wf1, bf1, Wf2, bf2)
